# baseline (speedup 1.0000x reference)
import functools

import jax
import jax.numpy as jnp
from jax import lax
from jax.experimental import pallas as pl
from jax.experimental.pallas import tpu as pltpu
from jax.experimental.pallas import tpu_sc as plsc

M_ITEM = 1000000
DIM = 32
B, S = 16384, 50
LANES = 128
N_ROWS = (B * S) // LANES
NW = 32
ROWS_PER_W = N_ROWS // NW
CH = 5
NCH = ROWS_PER_W // CH

NBLK_FULL = M_ITEM // LANES
NBLK_MAIN = (NBLK_FULL // NW) * NW
BLK_PER_W = NBLK_MAIN // NW
TAIL_ITEMS = M_ITEM - NBLK_FULL * LANES
B_BLOCKS = B // LANES


def _make_relayout():
    mesh = plsc.VectorSubcoreMesh(core_axis_name="c", subcore_axis_name="s")

    @functools.partial(
        pl.kernel,
        mesh=mesh,
        out_type=jax.ShapeDtypeStruct((M_ITEM * DIM // LANES, LANES),
                                      jnp.float32),
        scratch_types=[
            pltpu.VMEM((2, 4, 8, LANES), jnp.float32),
            pltpu.VMEM((2, DIM, LANES), jnp.float32),
            pltpu.SemaphoreType.DMA,
            pltpu.SemaphoreType.DMA,
            pltpu.SemaphoreType.DMA,
            pltpu.SemaphoreType.DMA,
        ],
        compiler_params=pltpu.CompilerParams(use_tc_tiling_on_sc=True, needs_layout_passes=False),
    )
    def relayout_kernel(emb_t, tail2, out, in_v, o_v, i0, i1, o0, o1):
        isem = (i0, i1)
        osem = (o0, o1)
        wid = lax.axis_index("s") * 2 + lax.axis_index("c")

        def fire_in(b, blk):
            for g in range(4):
                pltpu.async_copy(
                    emb_t.at[pl.ds(8 * g, 8), pl.ds(blk * LANES, LANES)],
                    in_v.at[b, g],
                    isem[b],
                )

        def drain_in(b):
            for g in range(4):
                pltpu.make_async_copy(
                    emb_t.at[pl.ds(0, 8), pl.ds(0, LANES)], in_v.at[b, g],
                    isem[b],
                ).wait()

        def transpose_block(b, n_items):
            lane = lax.iota(jnp.int32, 16)
            eight = jnp.full((16,), 8, jnp.int32)
            g_lo = lax.div(lane, eight)
            g_hi = g_lo + 2
            j_vec = lax.rem(lane, eight)
            for r in range(n_items):
                r_vec = jnp.full((16,), r, jnp.int32)
                lo = plsc.load_gather(in_v.at[b], [g_lo, j_vec, r_vec])
                hi = plsc.load_gather(in_v.at[b], [g_hi, j_vec, r_vec])
                p = r * DIM
                o_v[b, p // LANES, pl.ds(p % LANES, 16)] = lo
                o_v[b, p // LANES, pl.ds(p % LANES + 16, 16)] = hi

        def fire_out(b, blk):
            pltpu.async_copy(
                o_v.at[b], out.at[pl.ds(blk * DIM, DIM)], osem[b]
            )

        def wait_out(b):
            pltpu.make_async_copy(
                o_v.at[b], out.at[pl.ds(0, DIM)], osem[b]
            ).wait()

        fire_in(0, wid)
        fire_in(1, wid + NW)

        def body(k, carry):
            blk0 = wid + NW * (2 * k)
            for b in range(2):
                drain_in(b)
                transpose_block(b, LANES)
                fire_out(b, blk0 + NW * b)
                nxt = blk0 + NW * (b + 2)

                @pl.when(nxt < NBLK_MAIN)
                def _():
                    fire_in(b, nxt)

                wait_out(b)
            return carry

        lax.fori_loop(0, BLK_PER_W // 2, body, 0)

        @pl.when(wid < NBLK_FULL - NBLK_MAIN)
        def _():
            blk = NBLK_MAIN + wid
            fire_in(0, blk)
            drain_in(0)
            transpose_block(0, LANES)
            fire_out(0, blk)
            wait_out(0)

        @pl.when(wid == NBLK_FULL - NBLK_MAIN)
        def _():
            pltpu.sync_copy(tail2, out.at[pl.ds(NBLK_FULL * DIM, 16)])

    return relayout_kernel


def _make_gather():
    mesh = plsc.VectorSubcoreMesh(core_axis_name="c", subcore_axis_name="s")

    @functools.partial(
        pl.kernel,
        mesh=mesh,
        out_type=jax.ShapeDtypeStruct((S, 4, B_BLOCKS, 8, LANES), jnp.float32),
        scratch_types=[
            pltpu.VMEM((2, CH, LANES), jnp.int32),
            pltpu.VMEM((2, CH, LANES, DIM), jnp.float32),
            pltpu.VMEM((2, CH, 4, 8, LANES), jnp.float32),
            pltpu.SemaphoreType.DMA,
            pltpu.SemaphoreType.DMA,
            pltpu.SemaphoreType.DMA,
            pltpu.SemaphoreType.DMA,
        ],
        compiler_params=pltpu.CompilerParams(use_tc_tiling_on_sc=False, needs_layout_passes=False),
    )
    def gather_kernel(lin, idx_hbm, out, idx_v, rows_v, t_v, g0, g1, o0, o1):
        gsem = (g0, g1)
        osem = (o0, o1)
        wid = lax.axis_index("s") * 2 + lax.axis_index("c")
        base = wid * ROWS_PER_W
        lane = lax.iota(jnp.int32, 16)

        def fire(b, chunk):
            row = base + chunk * CH
            pltpu.sync_copy(idx_hbm.at[pl.ds(row, CH)], idx_v.at[b])
            for u in range(CH):
                pltpu.async_copy(
                    lin.at[idx_v.at[b].at[u]], rows_v.at[b].at[u], gsem[b]
                )

        def drain_gathers(b):
            for u in range(CH):
                pltpu.make_async_copy(
                    lin.at[pl.ds(0, LANES)], rows_v.at[b].at[u], gsem[b]
                ).wait()

        def transpose_chunk(b):
            def ubody(u, carry):
                lane = lax.iota(jnp.int32, 16)
                for g in range(4):
                    for jj in range(8):
                        d_vec = jnp.full((16,), 8 * g + jj, jnp.int32)
                        for k in range(8):
                            r_vec = lane + 16 * k
                            vals = plsc.load_gather(
                                rows_v.at[b, u], [r_vec, d_vec]
                            )
                            t_v[b, u, g, jj, pl.ds(16 * k, 16)] = vals
                return carry

            lax.fori_loop(0, CH, ubody, 0)

        def fire_out(b, chunk):
            for u in range(CH):
                m = base + chunk * CH + u
                s = m // B_BLOCKS
                cb = m % B_BLOCKS
                for g in range(4):
                    pltpu.async_copy(
                        t_v.at[b, u, g], out.at[s, g, cb], osem[b]
                    )

        def wait_out(b):
            for _ in range(4 * CH):
                pltpu.make_async_copy(
                    t_v.at[0, 0, 0], out.at[0, 0, 0], osem[b]
                ).wait()

        fire(0, 0)
        fire(1, 1)
        drain_gathers(0)
        transpose_chunk(0)
        fire_out(0, 0)

        def body(i, carry):
            c = 2 + 2 * i
            fire(0, c)
            drain_gathers(1)
            transpose_chunk(1)
            fire_out(1, c - 1)
            wait_out(0)
            fire(1, c + 1)
            drain_gathers(0)
            transpose_chunk(0)
            fire_out(0, c)
            wait_out(1)
            return carry

        lax.fori_loop(0, (NCH - 2) // 2, body, 0)

        drain_gathers(1)
        transpose_chunk(1)
        fire_out(1, NCH - 1)
        wait_out(0)
        wait_out(1)

    return gather_kernel


_relayout = _make_relayout()
_gather = _make_gather()


def kernel(indices, items_emb):
    emb_t = jnp.swapaxes(items_emb, 0, 1)
    tail2 = lax.slice(items_emb, (NBLK_FULL * LANES, 0), (M_ITEM, DIM))
    tail2 = tail2.reshape(16, LANES)
    lin = _relayout(emb_t, tail2).reshape(M_ITEM, DIM)
    idx2 = jnp.swapaxes(indices, 0, 1).reshape(N_ROWS, LANES).astype(jnp.int32)
    out5 = _gather(lin, idx2)
    return jnp.transpose(out5, (2, 4, 0, 1, 3)).reshape(B, S, DIM)
